# baseline (speedup 1.0000x reference)
import functools

import jax
import jax.numpy as jnp
from jax import lax
from jax.experimental import pallas as pl
from jax.experimental.pallas import tpu as pltpu
from jax.experimental.pallas import tpu_sc as plsc

N = 10000
E = 320000
IN = 128
HID = 32
OUT = 128

NC, NS = 2, 16
NW = NC * NS
CHUNK = 128
NCH = 80
EP = NW * NCH * CHUNK
PAD = EP - E
NB = 8
K = 2
ACC_N = 10112
RPT = 632

_MESH = plsc.VectorSubcoreMesh(core_axis_name="c", subcore_axis_name="s",
                               num_cores=NC, num_subcores=NS)
_SC_PARAMS = pltpu.CompilerParams(use_tc_tiling_on_sc=False)


def _sc_body(with_deg, feat_hbm, src_hbm, dst_hbm, zacc_hbm, zdeg_hbm, ones_hbm,
             acc_out, deg_out, src_v, dst_v, rows_v, ones_v,
             acc_sh, deg_sh, gsem, ssem, dsem):
    c = lax.axis_index("c")
    s = lax.axis_index("s")
    w = c * NS + s

    zrows = ACC_N // NS
    pltpu.sync_copy(zacc_hbm.at[pl.ds(s * zrows, zrows)],
                    acc_sh.at[pl.ds(s * zrows, zrows)])
    if with_deg:
        pltpu.sync_copy(zdeg_hbm.at[pl.ds(s * zrows, zrows)],
                        deg_sh.at[pl.ds(s * zrows, zrows)])
        pltpu.sync_copy(ones_hbm, ones_v)
    pltpu.sync_copy(src_hbm.at[w], src_v)
    pltpu.sync_copy(dst_hbm.at[w], dst_v)
    plsc.subcore_barrier()

    def gather(b, j):
        pltpu.async_copy(feat_hbm.at[src_v.at[j, 0]], rows_v.at[b],
                         gsem.at[b])

    for b in range(NB):
        gather(b, b)

    def group(g, carry):
        for b in range(NB):
            j = g * NB + b
            pltpu.make_async_copy(feat_hbm.at[src_v.at[j, 0]], rows_v.at[b],
                                  gsem.at[b]).wait()
            pltpu.async_copy(rows_v.at[b], acc_sh.at[dst_v.at[j, 0]],
                             ssem.at[b], add=True)
            if with_deg:
                pltpu.async_copy(ones_v, deg_sh.at[dst_v.at[j, 0]],
                                 dsem.at[b], add=True)
            bb = (b - K) % NB
            jj = j - K
            jn = j + NB - K

            @pl.when(jnp.logical_and(j >= K, jn < NCH))
            def _():
                pltpu.make_async_copy(rows_v.at[bb],
                                      acc_sh.at[dst_v.at[jj, 0]],
                                      ssem.at[bb]).wait()
                if with_deg:
                    pltpu.make_async_copy(ones_v,
                                          deg_sh.at[dst_v.at[jj, 0]],
                                          dsem.at[bb]).wait()
                gather(bb, jn)
        return carry

    lax.fori_loop(0, NCH // NB, group, 0)

    for b in range(NB):
        jd = NCH - NB + b
        pltpu.make_async_copy(rows_v.at[b], acc_sh.at[dst_v.at[jd, 0]],
                              ssem.at[b]).wait()
        if with_deg:
            pltpu.make_async_copy(ones_v, deg_sh.at[dst_v.at[jd, 0]],
                                  dsem.at[b]).wait()
    plsc.subcore_barrier()

    @pl.when(s < NS - 1)
    def _():
        pltpu.sync_copy(acc_sh.at[pl.ds(s * RPT, RPT)],
                        acc_out.at[c, pl.ds(s * RPT, RPT)])
        if with_deg:
            pltpu.sync_copy(deg_sh.at[pl.ds(s * RPT, RPT)],
                            deg_out.at[c, pl.ds(s * RPT, RPT)])

    @pl.when(s == NS - 1)
    def _():
        last = N - (NS - 1) * RPT
        pltpu.sync_copy(acc_sh.at[pl.ds((NS - 1) * RPT, last)],
                        acc_out.at[c, pl.ds((NS - 1) * RPT, last)])
        if with_deg:
            pltpu.sync_copy(deg_sh.at[pl.ds((NS - 1) * RPT, last)],
                            deg_out.at[c, pl.ds((NS - 1) * RPT, last)])


_SC_SCRATCH = [
    pltpu.VMEM((NCH, 1, CHUNK), jnp.int32),
    pltpu.VMEM((NCH, 1, CHUNK), jnp.int32),
    pltpu.VMEM((NB, CHUNK, HID), jnp.float32),
    pltpu.VMEM((CHUNK, 16), jnp.float32),
    pltpu.VMEM_SHARED((ACC_N, HID), jnp.float32),
    pltpu.VMEM_SHARED((ACC_N, 16), jnp.float32),
    pltpu.SemaphoreType.DMA((NB,)),
    pltpu.SemaphoreType.DMA((NB,)),
    pltpu.SemaphoreType.DMA((NB,)),
]

_sc_agg_deg = functools.partial(
    pl.kernel,
    out_type=(jax.ShapeDtypeStruct((NC, N, HID), jnp.float32),
              jax.ShapeDtypeStruct((NC, N, 16), jnp.float32)),
    mesh=_MESH,
    scratch_types=_SC_SCRATCH,
    compiler_params=_SC_PARAMS,
)(functools.partial(_sc_body, True))


def _sc_body_nodeg(feat_hbm, src_hbm, dst_hbm, zacc_hbm, acc_out,
                   src_v, dst_v, rows_v, acc_sh, gsem, ssem):
    _sc_body(False, feat_hbm, src_hbm, dst_hbm, zacc_hbm, None, None,
             acc_out, None, src_v, dst_v, rows_v, None,
             acc_sh, None, gsem, ssem, None)


_sc_agg = functools.partial(
    pl.kernel,
    out_type=jax.ShapeDtypeStruct((NC, N, HID), jnp.float32),
    mesh=_MESH,
    scratch_types=[
        pltpu.VMEM((NCH, 1, CHUNK), jnp.int32),
        pltpu.VMEM((NCH, 1, CHUNK), jnp.int32),
        pltpu.VMEM((NB, CHUNK, HID), jnp.float32),
        pltpu.VMEM_SHARED((ACC_N, HID), jnp.float32),
        pltpu.SemaphoreType.DMA((NB,)),
        pltpu.SemaphoreType.DMA((NB,)),
    ],
    compiler_params=_SC_PARAMS,
)(_sc_body_nodeg)


RB = 1000
GRID = N // RB


def _mm2_body(x_ref, wa_ref, wb_ref, oa_ref, ob_ref):
    x = x_ref[...]
    oa_ref[...] = jnp.dot(x, wa_ref[...], preferred_element_type=jnp.float32)
    ob_ref[...] = jnp.dot(x, wb_ref[...], preferred_element_type=jnp.float32)


def _tc_mm2(x, wa, wb):
    return pl.pallas_call(
        _mm2_body,
        grid=(GRID,),
        in_specs=[
            pl.BlockSpec((RB, IN), lambda i: (i, 0)),
            pl.BlockSpec((IN, HID), lambda i: (0, 0)),
            pl.BlockSpec((IN, HID), lambda i: (0, 0)),
        ],
        out_specs=[
            pl.BlockSpec((RB, HID), lambda i: (i, 0)),
            pl.BlockSpec((RB, HID), lambda i: (i, 0)),
        ],
        out_shape=[jax.ShapeDtypeStruct((N, HID), jnp.float32),
                   jax.ShapeDtypeStruct((N, HID), jnp.float32)],
    )(x, wa, wb)


def _combine(acc_ref, deg_ref):
    agg = acc_ref[0] + acc_ref[1]
    deg = deg_ref[0, :, 0:1] + deg_ref[1, :, 0:1]
    return agg / jnp.maximum(deg, 1.0)


def _layer_mid_body(s_ref, acc_ref, deg_ref, b_ref, ws_ref, wn_ref,
                    os_ref, op_ref):
    h = jnp.maximum(s_ref[...] + _combine(acc_ref, deg_ref) + b_ref[...], 0.0)
    os_ref[...] = jnp.dot(h, ws_ref[...], preferred_element_type=jnp.float32)
    op_ref[...] = jnp.dot(h, wn_ref[...], preferred_element_type=jnp.float32)


def _tc_layer_mid(sprev, acc, deg, b, ws, wn, ws_out_dim):
    return pl.pallas_call(
        _layer_mid_body,
        grid=(GRID,),
        in_specs=[
            pl.BlockSpec((RB, HID), lambda i: (i, 0)),
            pl.BlockSpec((NC, RB, HID), lambda i: (0, i, 0)),
            pl.BlockSpec((NC, RB, 16), lambda i: (0, i, 0)),
            pl.BlockSpec((1, HID), lambda i: (0, 0)),
            pl.BlockSpec((HID, ws_out_dim), lambda i: (0, 0)),
            pl.BlockSpec((HID, HID), lambda i: (0, 0)),
        ],
        out_specs=[
            pl.BlockSpec((RB, ws_out_dim), lambda i: (i, 0)),
            pl.BlockSpec((RB, HID), lambda i: (i, 0)),
        ],
        out_shape=[jax.ShapeDtypeStruct((N, ws_out_dim), jnp.float32),
                   jax.ShapeDtypeStruct((N, HID), jnp.float32)],
    )(sprev, acc, deg, b, ws, wn)


def _layer2_body(s_ref, acc_ref, deg_ref, b_ref, ws_ref, os_ref, oh_ref):
    h = jnp.maximum(s_ref[...] + _combine(acc_ref, deg_ref) + b_ref[...], 0.0)
    oh_ref[...] = h
    os_ref[...] = jnp.dot(h, ws_ref[...], preferred_element_type=jnp.float32)


def _tc_layer2(sprev, acc, deg, b, ws):
    return pl.pallas_call(
        _layer2_body,
        grid=(GRID,),
        in_specs=[
            pl.BlockSpec((RB, HID), lambda i: (i, 0)),
            pl.BlockSpec((NC, RB, HID), lambda i: (0, i, 0)),
            pl.BlockSpec((NC, RB, 16), lambda i: (0, i, 0)),
            pl.BlockSpec((1, HID), lambda i: (0, 0)),
            pl.BlockSpec((HID, OUT), lambda i: (0, 0)),
        ],
        out_specs=[
            pl.BlockSpec((RB, OUT), lambda i: (i, 0)),
            pl.BlockSpec((RB, HID), lambda i: (i, 0)),
        ],
        out_shape=[jax.ShapeDtypeStruct((N, OUT), jnp.float32),
                   jax.ShapeDtypeStruct((N, HID), jnp.float32)],
    )(sprev, acc, deg, b, ws)


def _layer3_body(s_ref, acc_ref, deg_ref, b_ref, wn_ref, o_ref):
    hn = _combine(acc_ref, deg_ref)
    o_ref[...] = (s_ref[...] + b_ref[...]
                  + jnp.dot(hn, wn_ref[...], preferred_element_type=jnp.float32))


def _tc_layer3(sprev, acc, deg, b, wn):
    return pl.pallas_call(
        _layer3_body,
        grid=(GRID,),
        in_specs=[
            pl.BlockSpec((RB, OUT), lambda i: (i, 0)),
            pl.BlockSpec((NC, RB, HID), lambda i: (0, i, 0)),
            pl.BlockSpec((NC, RB, 16), lambda i: (0, i, 0)),
            pl.BlockSpec((1, OUT), lambda i: (0, 0)),
            pl.BlockSpec((HID, OUT), lambda i: (0, 0)),
        ],
        out_specs=pl.BlockSpec((RB, OUT), lambda i: (i, 0)),
        out_shape=jax.ShapeDtypeStruct((N, OUT), jnp.float32),
    )(sprev, acc, deg, b, wn)


def kernel(x, edge_index, W_self1, W_neigh1, b1, W_self2, W_neigh2, b2,
           W_self3, W_neigh3, b3):
    src = jnp.concatenate(
        [edge_index[0].astype(jnp.int32), jnp.zeros((PAD,), jnp.int32)]
    ).reshape(NW, NCH, 1, CHUNK)
    dst = jnp.concatenate(
        [edge_index[1].astype(jnp.int32), jnp.full((PAD,), N, jnp.int32)]
    ).reshape(NW, NCH, 1, CHUNK)
    zacc = jnp.zeros((ACC_N, HID), jnp.float32)
    zdeg = jnp.zeros((ACC_N, 16), jnp.float32)
    ones = jnp.ones((CHUNK, 16), jnp.float32)

    s1, p1 = _tc_mm2(x, W_self1, W_neigh1)
    acc1, deg = _sc_agg_deg(p1, src, dst, zacc, zdeg, ones)
    s2, p2 = _tc_layer_mid(s1, acc1, deg, b1.reshape(1, HID),
                           W_self2, W_neigh2, HID)
    acc2 = _sc_agg(p2, src, dst, zacc)
    s3, h2 = _tc_layer2(s2, acc2, deg, b2.reshape(1, HID), W_self3)
    acc3 = _sc_agg(h2, src, dst, zacc)
    return _tc_layer3(s3, acc3, deg, b3.reshape(1, OUT), W_neigh3)


# speedup vs baseline: 1.0850x; 1.0850x over previous
import functools

import jax
import jax.numpy as jnp
from jax import lax
from jax.experimental import pallas as pl
from jax.experimental.pallas import tpu as pltpu
from jax.experimental.pallas import tpu_sc as plsc

N = 10000
E = 320000
IN = 128
HID = 32
OUT = 128

NC, NS = 2, 16
NW = NC * NS
CHUNK = 128
KK = 4
NGRP = 20
GEDGES = KK * CHUNK
EP = NW * NGRP * GEDGES
PAD = EP - E
ACC_N = 10112
RPT = 632

_MESH = plsc.VectorSubcoreMesh(core_axis_name="c", subcore_axis_name="s",
                               num_cores=NC, num_subcores=NS)
_SC_PARAMS = pltpu.CompilerParams(use_tc_tiling_on_sc=False)


def _sc_body(with_deg, feat_hbm, src_hbm, dst_hbm, zacc_hbm, zdeg_hbm, ones_hbm,
             acc_out, deg_out, src_v, dst_v, rows_v, ones_v,
             acc_sh, deg_sh, gsem):
    c = lax.axis_index("c")
    s = lax.axis_index("s")
    w = c * NS + s

    zrows = ACC_N // NS
    pltpu.sync_copy(zacc_hbm.at[pl.ds(s * zrows, zrows)],
                    acc_sh.at[pl.ds(s * zrows, zrows)])
    if with_deg:
        pltpu.sync_copy(zdeg_hbm.at[pl.ds(s * zrows, zrows)],
                        deg_sh.at[pl.ds(s * zrows, zrows)])
        pltpu.sync_copy(ones_hbm, ones_v)
    pltpu.sync_copy(src_hbm.at[w], src_v)
    pltpu.sync_copy(dst_hbm.at[w], dst_v)
    plsc.subcore_barrier()

    def gather(r, g):
        pltpu.async_copy(feat_hbm.at[src_v.at[g, 0]], rows_v.at[r],
                         gsem.at[r])

    gather(0, 0)

    def pair(p, carry):
        for r in range(2):
            g = 2 * p + r
            pltpu.make_async_copy(feat_hbm.at[src_v.at[g, 0]], rows_v.at[r],
                                  gsem.at[r]).wait()

            @pl.when(g + 1 < NGRP)
            def _():
                gather(1 - r, g + 1)

            pltpu.sync_copy(rows_v.at[r], acc_sh.at[dst_v.at[g, 0]], add=True)
            if with_deg:
                pltpu.sync_copy(ones_v, deg_sh.at[dst_v.at[g, 0]], add=True)
        return carry

    lax.fori_loop(0, NGRP // 2, pair, 0)
    plsc.subcore_barrier()

    @pl.when(s < NS - 1)
    def _():
        pltpu.sync_copy(acc_sh.at[pl.ds(s * RPT, RPT)],
                        acc_out.at[c, pl.ds(s * RPT, RPT)])
        if with_deg:
            pltpu.sync_copy(deg_sh.at[pl.ds(s * RPT, RPT)],
                            deg_out.at[c, pl.ds(s * RPT, RPT)])

    @pl.when(s == NS - 1)
    def _():
        last = N - (NS - 1) * RPT
        pltpu.sync_copy(acc_sh.at[pl.ds((NS - 1) * RPT, last)],
                        acc_out.at[c, pl.ds((NS - 1) * RPT, last)])
        if with_deg:
            pltpu.sync_copy(deg_sh.at[pl.ds((NS - 1) * RPT, last)],
                            deg_out.at[c, pl.ds((NS - 1) * RPT, last)])


_SC_SCRATCH = [
    pltpu.VMEM((NGRP, 1, GEDGES), jnp.int32),
    pltpu.VMEM((NGRP, 1, GEDGES), jnp.int32),
    pltpu.VMEM((2, GEDGES, HID), jnp.float32),
    pltpu.VMEM((GEDGES, 16), jnp.float32),
    pltpu.VMEM_SHARED((ACC_N, HID), jnp.float32),
    pltpu.VMEM_SHARED((ACC_N, 16), jnp.float32),
    pltpu.SemaphoreType.DMA((2,)),
]

_sc_agg_deg = functools.partial(
    pl.kernel,
    out_type=(jax.ShapeDtypeStruct((NC, N, HID), jnp.float32),
              jax.ShapeDtypeStruct((NC, N, 16), jnp.float32)),
    mesh=_MESH,
    scratch_types=_SC_SCRATCH,
    compiler_params=_SC_PARAMS,
)(functools.partial(_sc_body, True))


def _sc_body_nodeg(feat_hbm, src_hbm, dst_hbm, zacc_hbm, acc_out,
                   src_v, dst_v, rows_v, acc_sh, gsem):
    _sc_body(False, feat_hbm, src_hbm, dst_hbm, zacc_hbm, None, None,
             acc_out, None, src_v, dst_v, rows_v, None,
             acc_sh, None, gsem)


_sc_agg = functools.partial(
    pl.kernel,
    out_type=jax.ShapeDtypeStruct((NC, N, HID), jnp.float32),
    mesh=_MESH,
    scratch_types=[
        pltpu.VMEM((NGRP, 1, GEDGES), jnp.int32),
        pltpu.VMEM((NGRP, 1, GEDGES), jnp.int32),
        pltpu.VMEM((2, GEDGES, HID), jnp.float32),
        pltpu.VMEM_SHARED((ACC_N, HID), jnp.float32),
        pltpu.SemaphoreType.DMA((2,)),
    ],
    compiler_params=_SC_PARAMS,
)(_sc_body_nodeg)


RB = 1000
GRID = N // RB


def _mm2_body(x_ref, wa_ref, wb_ref, oa_ref, ob_ref):
    x = x_ref[...]
    oa_ref[...] = jnp.dot(x, wa_ref[...], preferred_element_type=jnp.float32)
    ob_ref[...] = jnp.dot(x, wb_ref[...], preferred_element_type=jnp.float32)


def _tc_mm2(x, wa, wb):
    return pl.pallas_call(
        _mm2_body,
        grid=(GRID,),
        in_specs=[
            pl.BlockSpec((RB, IN), lambda i: (i, 0)),
            pl.BlockSpec((IN, HID), lambda i: (0, 0)),
            pl.BlockSpec((IN, HID), lambda i: (0, 0)),
        ],
        out_specs=[
            pl.BlockSpec((RB, HID), lambda i: (i, 0)),
            pl.BlockSpec((RB, HID), lambda i: (i, 0)),
        ],
        out_shape=[jax.ShapeDtypeStruct((N, HID), jnp.float32),
                   jax.ShapeDtypeStruct((N, HID), jnp.float32)],
    )(x, wa, wb)


def _combine(acc_ref, deg_ref):
    agg = acc_ref[0] + acc_ref[1]
    deg = deg_ref[0, :, 0:1] + deg_ref[1, :, 0:1]
    return agg / jnp.maximum(deg, 1.0)


def _layer_mid_body(s_ref, acc_ref, deg_ref, b_ref, ws_ref, wn_ref,
                    os_ref, op_ref):
    h = jnp.maximum(s_ref[...] + _combine(acc_ref, deg_ref) + b_ref[...], 0.0)
    os_ref[...] = jnp.dot(h, ws_ref[...], preferred_element_type=jnp.float32)
    op_ref[...] = jnp.dot(h, wn_ref[...], preferred_element_type=jnp.float32)


def _tc_layer_mid(sprev, acc, deg, b, ws, wn, ws_out_dim):
    return pl.pallas_call(
        _layer_mid_body,
        grid=(GRID,),
        in_specs=[
            pl.BlockSpec((RB, HID), lambda i: (i, 0)),
            pl.BlockSpec((NC, RB, HID), lambda i: (0, i, 0)),
            pl.BlockSpec((NC, RB, 16), lambda i: (0, i, 0)),
            pl.BlockSpec((1, HID), lambda i: (0, 0)),
            pl.BlockSpec((HID, ws_out_dim), lambda i: (0, 0)),
            pl.BlockSpec((HID, HID), lambda i: (0, 0)),
        ],
        out_specs=[
            pl.BlockSpec((RB, ws_out_dim), lambda i: (i, 0)),
            pl.BlockSpec((RB, HID), lambda i: (i, 0)),
        ],
        out_shape=[jax.ShapeDtypeStruct((N, ws_out_dim), jnp.float32),
                   jax.ShapeDtypeStruct((N, HID), jnp.float32)],
    )(sprev, acc, deg, b, ws, wn)


def _layer2_body(s_ref, acc_ref, deg_ref, b_ref, ws_ref, os_ref, oh_ref):
    h = jnp.maximum(s_ref[...] + _combine(acc_ref, deg_ref) + b_ref[...], 0.0)
    oh_ref[...] = h
    os_ref[...] = jnp.dot(h, ws_ref[...], preferred_element_type=jnp.float32)


def _tc_layer2(sprev, acc, deg, b, ws):
    return pl.pallas_call(
        _layer2_body,
        grid=(GRID,),
        in_specs=[
            pl.BlockSpec((RB, HID), lambda i: (i, 0)),
            pl.BlockSpec((NC, RB, HID), lambda i: (0, i, 0)),
            pl.BlockSpec((NC, RB, 16), lambda i: (0, i, 0)),
            pl.BlockSpec((1, HID), lambda i: (0, 0)),
            pl.BlockSpec((HID, OUT), lambda i: (0, 0)),
        ],
        out_specs=[
            pl.BlockSpec((RB, OUT), lambda i: (i, 0)),
            pl.BlockSpec((RB, HID), lambda i: (i, 0)),
        ],
        out_shape=[jax.ShapeDtypeStruct((N, OUT), jnp.float32),
                   jax.ShapeDtypeStruct((N, HID), jnp.float32)],
    )(sprev, acc, deg, b, ws)


def _layer3_body(s_ref, acc_ref, deg_ref, b_ref, wn_ref, o_ref):
    hn = _combine(acc_ref, deg_ref)
    o_ref[...] = (s_ref[...] + b_ref[...]
                  + jnp.dot(hn, wn_ref[...], preferred_element_type=jnp.float32))


def _tc_layer3(sprev, acc, deg, b, wn):
    return pl.pallas_call(
        _layer3_body,
        grid=(GRID,),
        in_specs=[
            pl.BlockSpec((RB, OUT), lambda i: (i, 0)),
            pl.BlockSpec((NC, RB, HID), lambda i: (0, i, 0)),
            pl.BlockSpec((NC, RB, 16), lambda i: (0, i, 0)),
            pl.BlockSpec((1, OUT), lambda i: (0, 0)),
            pl.BlockSpec((HID, OUT), lambda i: (0, 0)),
        ],
        out_specs=pl.BlockSpec((RB, OUT), lambda i: (i, 0)),
        out_shape=jax.ShapeDtypeStruct((N, OUT), jnp.float32),
    )(sprev, acc, deg, b, wn)


def kernel(x, edge_index, W_self1, W_neigh1, b1, W_self2, W_neigh2, b2,
           W_self3, W_neigh3, b3):
    src = jnp.concatenate(
        [edge_index[0].astype(jnp.int32), jnp.zeros((PAD,), jnp.int32)]
    ).reshape(NW, NGRP, 1, GEDGES)
    dst = jnp.concatenate(
        [edge_index[1].astype(jnp.int32), jnp.full((PAD,), N, jnp.int32)]
    ).reshape(NW, NGRP, 1, GEDGES)
    zacc = jnp.zeros((ACC_N, HID), jnp.float32)
    zdeg = jnp.zeros((ACC_N, 16), jnp.float32)
    ones = jnp.ones((GEDGES, 16), jnp.float32)

    s1, p1 = _tc_mm2(x, W_self1, W_neigh1)
    acc1, deg = _sc_agg_deg(p1, src, dst, zacc, zdeg, ones)
    s2, p2 = _tc_layer_mid(s1, acc1, deg, b1.reshape(1, HID),
                           W_self2, W_neigh2, HID)
    acc2 = _sc_agg(p2, src, dst, zacc)
    s3, h2 = _tc_layer2(s2, acc2, deg, b2.reshape(1, HID), W_self3)
    acc3 = _sc_agg(h2, src, dst, zacc)
    return _tc_layer3(s3, acc3, deg, b3.reshape(1, OUT), W_neigh3)


# speedup vs baseline: 2.0283x; 1.8695x over previous
import functools

import jax
import jax.numpy as jnp
from jax import lax
from jax.experimental import pallas as pl
from jax.experimental.pallas import tpu as pltpu
from jax.experimental.pallas import tpu_sc as plsc

N = 10000
E = 320000
IN = 128
HID = 32
OUT = 128

NC, NS = 2, 16
NW = NC * NS
EPW = E // NW
GEDGES = 200
NGRP = EPW // GEDGES
ACC_N = 10112
RPT = 632
ZR = ACC_N // NS

_MESH = plsc.VectorSubcoreMesh(core_axis_name="c", subcore_axis_name="s",
                               num_cores=NC, num_subcores=NS)
_SC_PARAMS = pltpu.CompilerParams(use_tc_tiling_on_sc=False)


def _sc_body(with_deg, feat_hbm, src_hbm, dst_hbm,
             acc_out, deg_out, src_v, dst_v, rows_v, ones_v,
             acc_sh, deg_sh, gsem, isem):
    c = lax.axis_index("c")
    s = lax.axis_index("s")
    w = c * NS + s
    base = w * EPW

    pltpu.async_copy(src_hbm.at[pl.ds(base, EPW)], src_v, isem)
    for g in range(NGRP):
        pltpu.async_copy(dst_hbm.at[pl.ds(base + g * GEDGES, GEDGES)],
                         dst_v.at[g, 0], gsem.at[0])

    zv = jnp.zeros((16,), jnp.float32)

    def zrow(i, carry):
        rows_v[0, i, 0:16] = zv
        rows_v[0, i, 16:32] = zv
        rows_v[1, i, 0:16] = zv
        rows_v[1, i, 16:32] = zv
        if with_deg:
            ones_v[i] = zv
        return carry

    lax.fori_loop(0, GEDGES, zrow, 0)
    for q in range(3):
        pltpu.sync_copy(rows_v.at[q % 2],
                        acc_sh.at[pl.ds(s * ZR + q * GEDGES, GEDGES)])
    pltpu.sync_copy(rows_v.at[0, pl.ds(0, ZR - 3 * GEDGES)],
                    acc_sh.at[pl.ds(s * ZR + 3 * GEDGES, ZR - 3 * GEDGES)])
    if with_deg:
        for q in range(3):
            pltpu.sync_copy(ones_v,
                            deg_sh.at[pl.ds(s * ZR + q * GEDGES, GEDGES)])
        pltpu.sync_copy(ones_v.at[pl.ds(0, ZR - 3 * GEDGES)],
                        deg_sh.at[pl.ds(s * ZR + 3 * GEDGES, ZR - 3 * GEDGES)])
        ov = jnp.ones((16,), jnp.float32)

        def orow(i, carry):
            ones_v[i] = ov
            return carry

        lax.fori_loop(0, GEDGES, orow, 0)

    pltpu.make_async_copy(src_hbm.at[pl.ds(base, EPW)], src_v, isem).wait()
    for g in range(NGRP):
        pltpu.make_async_copy(dst_hbm.at[pl.ds(base + g * GEDGES, GEDGES)],
                              dst_v.at[g, 0], gsem.at[0]).wait()
    plsc.subcore_barrier()

    def gather(r, g):
        pltpu.async_copy(feat_hbm.at[src_v.at[pl.ds(g * GEDGES, GEDGES)]],
                         rows_v.at[r], gsem.at[r])

    gather(0, 0)

    def pair(p, carry):
        for r in range(2):
            g = 2 * p + r
            pltpu.make_async_copy(
                feat_hbm.at[src_v.at[pl.ds(g * GEDGES, GEDGES)]],
                rows_v.at[r], gsem.at[r]).wait()

            @pl.when(g + 1 < NGRP)
            def _():
                gather(1 - r, g + 1)

            pltpu.sync_copy(rows_v.at[r], acc_sh.at[dst_v.at[g, 0]], add=True)
            if with_deg:
                pltpu.sync_copy(ones_v, deg_sh.at[dst_v.at[g, 0]], add=True)
        return carry

    lax.fori_loop(0, NGRP // 2, pair, 0)
    plsc.subcore_barrier()

    @pl.when(s < NS - 1)
    def _():
        pltpu.sync_copy(acc_sh.at[pl.ds(s * RPT, RPT)],
                        acc_out.at[c, pl.ds(s * RPT, RPT)])
        if with_deg:
            pltpu.sync_copy(deg_sh.at[pl.ds(s * RPT, RPT)],
                            deg_out.at[c, pl.ds(s * RPT, RPT)])

    @pl.when(s == NS - 1)
    def _():
        last = N - (NS - 1) * RPT
        pltpu.sync_copy(acc_sh.at[pl.ds((NS - 1) * RPT, last)],
                        acc_out.at[c, pl.ds((NS - 1) * RPT, last)])
        if with_deg:
            pltpu.sync_copy(deg_sh.at[pl.ds((NS - 1) * RPT, last)],
                            deg_out.at[c, pl.ds((NS - 1) * RPT, last)])


_sc_agg_deg = functools.partial(
    pl.kernel,
    out_type=(jax.ShapeDtypeStruct((NC, N, HID), jnp.float32),
              jax.ShapeDtypeStruct((NC, N, 16), jnp.float32)),
    mesh=_MESH,
    scratch_types=[
        pltpu.VMEM((EPW,), jnp.int32),
        pltpu.VMEM((NGRP, 1, GEDGES), jnp.int32),
        pltpu.VMEM((2, GEDGES, HID), jnp.float32),
        pltpu.VMEM((GEDGES, 16), jnp.float32),
        pltpu.VMEM_SHARED((ACC_N, HID), jnp.float32),
        pltpu.VMEM_SHARED((ACC_N, 16), jnp.float32),
        pltpu.SemaphoreType.DMA((2,)),
        pltpu.SemaphoreType.DMA,
    ],
    compiler_params=_SC_PARAMS,
)(functools.partial(_sc_body, True))


def _sc_body_nodeg(feat_hbm, src_hbm, dst_hbm, acc_out,
                   src_v, dst_v, rows_v, acc_sh, gsem, isem):
    _sc_body(False, feat_hbm, src_hbm, dst_hbm,
             acc_out, None, src_v, dst_v, rows_v, None,
             acc_sh, None, gsem, isem)


_sc_agg = functools.partial(
    pl.kernel,
    out_type=jax.ShapeDtypeStruct((NC, N, HID), jnp.float32),
    mesh=_MESH,
    scratch_types=[
        pltpu.VMEM((EPW,), jnp.int32),
        pltpu.VMEM((NGRP, 1, GEDGES), jnp.int32),
        pltpu.VMEM((2, GEDGES, HID), jnp.float32),
        pltpu.VMEM_SHARED((ACC_N, HID), jnp.float32),
        pltpu.SemaphoreType.DMA((2,)),
        pltpu.SemaphoreType.DMA,
    ],
    compiler_params=_SC_PARAMS,
)(_sc_body_nodeg)


RB = 2000
GRID = N // RB


def _mm2_body(x_ref, wa_ref, wb_ref, oa_ref, ob_ref):
    x = x_ref[...]
    oa_ref[...] = jnp.dot(x, wa_ref[...], preferred_element_type=jnp.float32)
    ob_ref[...] = jnp.dot(x, wb_ref[...], preferred_element_type=jnp.float32)


def _tc_mm2(x, wa, wb):
    return pl.pallas_call(
        _mm2_body,
        grid=(GRID,),
        in_specs=[
            pl.BlockSpec((RB, IN), lambda i: (i, 0)),
            pl.BlockSpec((IN, HID), lambda i: (0, 0)),
            pl.BlockSpec((IN, HID), lambda i: (0, 0)),
        ],
        out_specs=[
            pl.BlockSpec((RB, HID), lambda i: (i, 0)),
            pl.BlockSpec((RB, HID), lambda i: (i, 0)),
        ],
        out_shape=[jax.ShapeDtypeStruct((N, HID), jnp.float32),
                   jax.ShapeDtypeStruct((N, HID), jnp.float32)],
    )(x, wa, wb)


def _combine(acc_ref, deg_ref):
    agg = acc_ref[0] + acc_ref[1]
    deg = deg_ref[0, :, 0:1] + deg_ref[1, :, 0:1]
    return agg / jnp.maximum(deg, 1.0)


def _layer_mid_body(s_ref, acc_ref, deg_ref, b_ref, ws_ref, wn_ref,
                    os_ref, op_ref):
    h = jnp.maximum(s_ref[...] + _combine(acc_ref, deg_ref) + b_ref[...], 0.0)
    os_ref[...] = jnp.dot(h, ws_ref[...], preferred_element_type=jnp.float32)
    op_ref[...] = jnp.dot(h, wn_ref[...], preferred_element_type=jnp.float32)


def _tc_layer_mid(sprev, acc, deg, b, ws, wn, ws_out_dim):
    return pl.pallas_call(
        _layer_mid_body,
        grid=(GRID,),
        in_specs=[
            pl.BlockSpec((RB, HID), lambda i: (i, 0)),
            pl.BlockSpec((NC, RB, HID), lambda i: (0, i, 0)),
            pl.BlockSpec((NC, RB, 16), lambda i: (0, i, 0)),
            pl.BlockSpec((1, HID), lambda i: (0, 0)),
            pl.BlockSpec((HID, ws_out_dim), lambda i: (0, 0)),
            pl.BlockSpec((HID, HID), lambda i: (0, 0)),
        ],
        out_specs=[
            pl.BlockSpec((RB, ws_out_dim), lambda i: (i, 0)),
            pl.BlockSpec((RB, HID), lambda i: (i, 0)),
        ],
        out_shape=[jax.ShapeDtypeStruct((N, ws_out_dim), jnp.float32),
                   jax.ShapeDtypeStruct((N, HID), jnp.float32)],
    )(sprev, acc, deg, b, ws, wn)


def _layer2_body(s_ref, acc_ref, deg_ref, b_ref, ws_ref, os_ref, oh_ref):
    h = jnp.maximum(s_ref[...] + _combine(acc_ref, deg_ref) + b_ref[...], 0.0)
    oh_ref[...] = h
    os_ref[...] = jnp.dot(h, ws_ref[...], preferred_element_type=jnp.float32)


def _tc_layer2(sprev, acc, deg, b, ws):
    return pl.pallas_call(
        _layer2_body,
        grid=(GRID,),
        in_specs=[
            pl.BlockSpec((RB, HID), lambda i: (i, 0)),
            pl.BlockSpec((NC, RB, HID), lambda i: (0, i, 0)),
            pl.BlockSpec((NC, RB, 16), lambda i: (0, i, 0)),
            pl.BlockSpec((1, HID), lambda i: (0, 0)),
            pl.BlockSpec((HID, OUT), lambda i: (0, 0)),
        ],
        out_specs=[
            pl.BlockSpec((RB, OUT), lambda i: (i, 0)),
            pl.BlockSpec((RB, HID), lambda i: (i, 0)),
        ],
        out_shape=[jax.ShapeDtypeStruct((N, OUT), jnp.float32),
                   jax.ShapeDtypeStruct((N, HID), jnp.float32)],
    )(sprev, acc, deg, b, ws)


def _layer3_body(s_ref, acc_ref, deg_ref, b_ref, wn_ref, o_ref):
    hn = _combine(acc_ref, deg_ref)
    o_ref[...] = (s_ref[...] + b_ref[...]
                  + jnp.dot(hn, wn_ref[...], preferred_element_type=jnp.float32))


def _tc_layer3(sprev, acc, deg, b, wn):
    return pl.pallas_call(
        _layer3_body,
        grid=(GRID,),
        in_specs=[
            pl.BlockSpec((RB, OUT), lambda i: (i, 0)),
            pl.BlockSpec((NC, RB, HID), lambda i: (0, i, 0)),
            pl.BlockSpec((NC, RB, 16), lambda i: (0, i, 0)),
            pl.BlockSpec((1, OUT), lambda i: (0, 0)),
            pl.BlockSpec((HID, OUT), lambda i: (0, 0)),
        ],
        out_specs=pl.BlockSpec((RB, OUT), lambda i: (i, 0)),
        out_shape=jax.ShapeDtypeStruct((N, OUT), jnp.float32),
    )(sprev, acc, deg, b, wn)


def kernel(x, edge_index, W_self1, W_neigh1, b1, W_self2, W_neigh2, b2,
           W_self3, W_neigh3, b3):
    src = edge_index[0].astype(jnp.int32)
    dst = edge_index[1].astype(jnp.int32)

    s1, p1 = _tc_mm2(x, W_self1, W_neigh1)
    acc1, deg = _sc_agg_deg(p1, src, dst)
    s2, p2 = _tc_layer_mid(s1, acc1, deg, b1.reshape(1, HID),
                           W_self2, W_neigh2, HID)
    acc2 = _sc_agg(p2, src, dst)
    s3, h2 = _tc_layer2(s2, acc2, deg, b2.reshape(1, HID), W_self3)
    acc3 = _sc_agg(h2, src, dst)
    return _tc_layer3(s3, acc3, deg, b3.reshape(1, OUT), W_neigh3)


# speedup vs baseline: 2.1048x; 1.0377x over previous
import functools

import jax
import jax.numpy as jnp
from jax import lax
from jax.experimental import pallas as pl
from jax.experimental.pallas import tpu as pltpu
from jax.experimental.pallas import tpu_sc as plsc

N = 10000
E = 320000
IN = 128
HID = 32
OUT = 128

NC, NS = 2, 16
NW = NC * NS
EPW = E // NW
GEDGES = 200
NGRP = EPW // GEDGES
ACC_N = 10112
RPT = 632
ZR = ACC_N // NS

_MESH = plsc.VectorSubcoreMesh(core_axis_name="c", subcore_axis_name="s",
                               num_cores=NC, num_subcores=NS)
_SC_PARAMS = pltpu.CompilerParams(use_tc_tiling_on_sc=False)


def _sc_body(with_deg, feat_hbm, edge_hbm,
             acc_out, deg_out, src_v, dst_v, rows_v, ones_v,
             acc_sh, deg_sh, gsem, isem):
    c = lax.axis_index("c")
    s = lax.axis_index("s")
    w = c * NS + s
    base = w * EPW

    pltpu.async_copy(edge_hbm.at[0, pl.ds(base, EPW)], src_v, isem)
    for g in range(NGRP):
        pltpu.async_copy(edge_hbm.at[1, pl.ds(base + g * GEDGES, GEDGES)],
                         dst_v.at[g, 0], gsem.at[0])

    zv = jnp.zeros((16,), jnp.float32)

    def zrow(i, carry):
        rows_v[0, i, 0:16] = zv
        rows_v[0, i, 16:32] = zv
        rows_v[1, i, 0:16] = zv
        rows_v[1, i, 16:32] = zv
        if with_deg:
            ones_v[i] = zv
        return carry

    lax.fori_loop(0, GEDGES, zrow, 0)
    for q in range(3):
        pltpu.sync_copy(rows_v.at[q % 2],
                        acc_sh.at[pl.ds(s * ZR + q * GEDGES, GEDGES)])
    pltpu.sync_copy(rows_v.at[0, pl.ds(0, ZR - 3 * GEDGES)],
                    acc_sh.at[pl.ds(s * ZR + 3 * GEDGES, ZR - 3 * GEDGES)])
    if with_deg:
        for q in range(3):
            pltpu.sync_copy(ones_v,
                            deg_sh.at[pl.ds(s * ZR + q * GEDGES, GEDGES)])
        pltpu.sync_copy(ones_v.at[pl.ds(0, ZR - 3 * GEDGES)],
                        deg_sh.at[pl.ds(s * ZR + 3 * GEDGES, ZR - 3 * GEDGES)])
        ov = jnp.ones((16,), jnp.float32)

        def orow(i, carry):
            ones_v[i] = ov
            return carry

        lax.fori_loop(0, GEDGES, orow, 0)

    pltpu.make_async_copy(edge_hbm.at[0, pl.ds(base, EPW)], src_v,
                          isem).wait()
    for g in range(NGRP):
        pltpu.make_async_copy(edge_hbm.at[1, pl.ds(base + g * GEDGES, GEDGES)],
                              dst_v.at[g, 0], gsem.at[0]).wait()
    plsc.subcore_barrier()

    def gather(r, g):
        pltpu.async_copy(feat_hbm.at[src_v.at[pl.ds(g * GEDGES, GEDGES)]],
                         rows_v.at[r], gsem.at[r])

    gather(0, 0)

    def pair(p, carry):
        for r in range(2):
            g = 2 * p + r
            pltpu.make_async_copy(
                feat_hbm.at[src_v.at[pl.ds(g * GEDGES, GEDGES)]],
                rows_v.at[r], gsem.at[r]).wait()

            @pl.when(g + 1 < NGRP)
            def _():
                gather(1 - r, g + 1)

            pltpu.sync_copy(rows_v.at[r], acc_sh.at[dst_v.at[g, 0]], add=True)
            if with_deg:
                pltpu.sync_copy(ones_v, deg_sh.at[dst_v.at[g, 0]], add=True)
        return carry

    lax.fori_loop(0, NGRP // 2, pair, 0)
    plsc.subcore_barrier()

    @pl.when(s < NS - 1)
    def _():
        pltpu.sync_copy(acc_sh.at[pl.ds(s * RPT, RPT)],
                        acc_out.at[c, pl.ds(s * RPT, RPT)])
        if with_deg:
            pltpu.sync_copy(deg_sh.at[pl.ds(s * RPT, RPT)],
                            deg_out.at[c, pl.ds(s * RPT, RPT)])

    @pl.when(s == NS - 1)
    def _():
        last = N - (NS - 1) * RPT
        pltpu.sync_copy(acc_sh.at[pl.ds((NS - 1) * RPT, last)],
                        acc_out.at[c, pl.ds((NS - 1) * RPT, last)])
        if with_deg:
            pltpu.sync_copy(deg_sh.at[pl.ds((NS - 1) * RPT, last)],
                            deg_out.at[c, pl.ds((NS - 1) * RPT, last)])


_sc_agg_deg = functools.partial(
    pl.kernel,
    out_type=(jax.ShapeDtypeStruct((NC, N, HID), jnp.float32),
              jax.ShapeDtypeStruct((NC, N, 16), jnp.float32)),
    mesh=_MESH,
    scratch_types=[
        pltpu.VMEM((EPW,), jnp.int32),
        pltpu.VMEM((NGRP, 1, GEDGES), jnp.int32),
        pltpu.VMEM((2, GEDGES, HID), jnp.float32),
        pltpu.VMEM((GEDGES, 16), jnp.float32),
        pltpu.VMEM_SHARED((ACC_N, HID), jnp.float32),
        pltpu.VMEM_SHARED((ACC_N, 16), jnp.float32),
        pltpu.SemaphoreType.DMA((2,)),
        pltpu.SemaphoreType.DMA,
    ],
    compiler_params=_SC_PARAMS,
)(functools.partial(_sc_body, True))


def _sc_body_nodeg(feat_hbm, edge_hbm, acc_out,
                   src_v, dst_v, rows_v, acc_sh, gsem, isem):
    _sc_body(False, feat_hbm, edge_hbm,
             acc_out, None, src_v, dst_v, rows_v, None,
             acc_sh, None, gsem, isem)


_sc_agg = functools.partial(
    pl.kernel,
    out_type=jax.ShapeDtypeStruct((NC, N, HID), jnp.float32),
    mesh=_MESH,
    scratch_types=[
        pltpu.VMEM((EPW,), jnp.int32),
        pltpu.VMEM((NGRP, 1, GEDGES), jnp.int32),
        pltpu.VMEM((2, GEDGES, HID), jnp.float32),
        pltpu.VMEM_SHARED((ACC_N, HID), jnp.float32),
        pltpu.SemaphoreType.DMA((2,)),
        pltpu.SemaphoreType.DMA,
    ],
    compiler_params=_SC_PARAMS,
)(_sc_body_nodeg)


RB = 2000
GRID = N // RB


def _mm2_body(x_ref, wa_ref, wb_ref, oa_ref, ob_ref):
    x = x_ref[...]
    oa_ref[...] = jnp.dot(x, wa_ref[...], preferred_element_type=jnp.float32)
    ob_ref[...] = jnp.dot(x, wb_ref[...], preferred_element_type=jnp.float32)


def _tc_mm2(x, wa, wb):
    return pl.pallas_call(
        _mm2_body,
        grid=(GRID,),
        in_specs=[
            pl.BlockSpec((RB, IN), lambda i: (i, 0)),
            pl.BlockSpec((IN, HID), lambda i: (0, 0)),
            pl.BlockSpec((IN, HID), lambda i: (0, 0)),
        ],
        out_specs=[
            pl.BlockSpec((RB, HID), lambda i: (i, 0)),
            pl.BlockSpec((RB, HID), lambda i: (i, 0)),
        ],
        out_shape=[jax.ShapeDtypeStruct((N, HID), jnp.float32),
                   jax.ShapeDtypeStruct((N, HID), jnp.float32)],
    )(x, wa, wb)


def _combine(acc_ref, deg_ref):
    agg = acc_ref[0] + acc_ref[1]
    deg = deg_ref[0, :, 0:1] + deg_ref[1, :, 0:1]
    return agg / jnp.maximum(deg, 1.0)


def _layer_mid_body(s_ref, acc_ref, deg_ref, b_ref, ws_ref, wn_ref,
                    os_ref, op_ref):
    h = jnp.maximum(s_ref[...] + _combine(acc_ref, deg_ref) + b_ref[...], 0.0)
    os_ref[...] = jnp.dot(h, ws_ref[...], preferred_element_type=jnp.float32)
    op_ref[...] = jnp.dot(h, wn_ref[...], preferred_element_type=jnp.float32)


def _tc_layer_mid(sprev, acc, deg, b, ws, wn, ws_out_dim):
    return pl.pallas_call(
        _layer_mid_body,
        grid=(GRID,),
        in_specs=[
            pl.BlockSpec((RB, HID), lambda i: (i, 0)),
            pl.BlockSpec((NC, RB, HID), lambda i: (0, i, 0)),
            pl.BlockSpec((NC, RB, 16), lambda i: (0, i, 0)),
            pl.BlockSpec((1, HID), lambda i: (0, 0)),
            pl.BlockSpec((HID, ws_out_dim), lambda i: (0, 0)),
            pl.BlockSpec((HID, HID), lambda i: (0, 0)),
        ],
        out_specs=[
            pl.BlockSpec((RB, ws_out_dim), lambda i: (i, 0)),
            pl.BlockSpec((RB, HID), lambda i: (i, 0)),
        ],
        out_shape=[jax.ShapeDtypeStruct((N, ws_out_dim), jnp.float32),
                   jax.ShapeDtypeStruct((N, HID), jnp.float32)],
    )(sprev, acc, deg, b, ws, wn)


def _layer2_body(s_ref, acc_ref, deg_ref, b_ref, ws_ref, os_ref, oh_ref):
    h = jnp.maximum(s_ref[...] + _combine(acc_ref, deg_ref) + b_ref[...], 0.0)
    oh_ref[...] = h
    os_ref[...] = jnp.dot(h, ws_ref[...], preferred_element_type=jnp.float32)


def _tc_layer2(sprev, acc, deg, b, ws):
    return pl.pallas_call(
        _layer2_body,
        grid=(GRID,),
        in_specs=[
            pl.BlockSpec((RB, HID), lambda i: (i, 0)),
            pl.BlockSpec((NC, RB, HID), lambda i: (0, i, 0)),
            pl.BlockSpec((NC, RB, 16), lambda i: (0, i, 0)),
            pl.BlockSpec((1, HID), lambda i: (0, 0)),
            pl.BlockSpec((HID, OUT), lambda i: (0, 0)),
        ],
        out_specs=[
            pl.BlockSpec((RB, OUT), lambda i: (i, 0)),
            pl.BlockSpec((RB, HID), lambda i: (i, 0)),
        ],
        out_shape=[jax.ShapeDtypeStruct((N, OUT), jnp.float32),
                   jax.ShapeDtypeStruct((N, HID), jnp.float32)],
    )(sprev, acc, deg, b, ws)


def _layer3_body(s_ref, acc_ref, deg_ref, b_ref, wn_ref, o_ref):
    hn = _combine(acc_ref, deg_ref)
    o_ref[...] = (s_ref[...] + b_ref[...]
                  + jnp.dot(hn, wn_ref[...], preferred_element_type=jnp.float32))


def _tc_layer3(sprev, acc, deg, b, wn):
    return pl.pallas_call(
        _layer3_body,
        grid=(GRID,),
        in_specs=[
            pl.BlockSpec((RB, OUT), lambda i: (i, 0)),
            pl.BlockSpec((NC, RB, HID), lambda i: (0, i, 0)),
            pl.BlockSpec((NC, RB, 16), lambda i: (0, i, 0)),
            pl.BlockSpec((1, OUT), lambda i: (0, 0)),
            pl.BlockSpec((HID, OUT), lambda i: (0, 0)),
        ],
        out_specs=pl.BlockSpec((RB, OUT), lambda i: (i, 0)),
        out_shape=jax.ShapeDtypeStruct((N, OUT), jnp.float32),
    )(sprev, acc, deg, b, wn)


def kernel(x, edge_index, W_self1, W_neigh1, b1, W_self2, W_neigh2, b2,
           W_self3, W_neigh3, b3):
    edges = edge_index.astype(jnp.int32)

    s1, p1 = _tc_mm2(x, W_self1, W_neigh1)
    acc1, deg = _sc_agg_deg(p1, edges)
    s2, p2 = _tc_layer_mid(s1, acc1, deg, b1.reshape(1, HID),
                           W_self2, W_neigh2, HID)
    acc2 = _sc_agg(p2, edges)
    s3, h2 = _tc_layer2(s2, acc2, deg, b2.reshape(1, HID), W_self3)
    acc3 = _sc_agg(h2, edges)
    return _tc_layer3(s3, acc3, deg, b3.reshape(1, OUT), W_neigh3)


# speedup vs baseline: 2.1135x; 1.0041x over previous
import functools

import jax
import jax.numpy as jnp
from jax import lax
from jax.experimental import pallas as pl
from jax.experimental.pallas import tpu as pltpu
from jax.experimental.pallas import tpu_sc as plsc

N = 10000
E = 320000
IN = 128
HID = 32
OUT = 128

NC, NS = 2, 16
NW = NC * NS
EPW = E // NW
GEDGES = 200
NGRP = EPW // GEDGES
ACC_N = 10112
RPT = 632
ZR = ACC_N // NS

_MESH = plsc.VectorSubcoreMesh(core_axis_name="c", subcore_axis_name="s",
                               num_cores=NC, num_subcores=NS)
_SC_PARAMS = pltpu.CompilerParams(use_tc_tiling_on_sc=False)


def _sc_body(with_deg, feat_hbm, edge_hbm,
             acc_out, deg_out, src_v, dst_v, rows_v, ones_v,
             acc_sh, deg_sh, gsem, isem):
    c = lax.axis_index("c")
    s = lax.axis_index("s")
    w = c * NS + s
    base = w * EPW

    pltpu.async_copy(edge_hbm.at[0, pl.ds(base, EPW)], src_v, isem)
    for g in range(NGRP):
        pltpu.async_copy(edge_hbm.at[1, pl.ds(base + g * GEDGES, GEDGES)],
                         dst_v.at[g, 0], gsem.at[0])

    zv = jnp.zeros((16,), jnp.float32)

    def zrow(i, carry):
        rows_v[0, i, 0:16] = zv
        rows_v[0, i, 16:32] = zv
        rows_v[1, i, 0:16] = zv
        rows_v[1, i, 16:32] = zv
        if with_deg:
            ones_v[i] = zv
        return carry

    lax.fori_loop(0, GEDGES, zrow, 0)
    for q in range(3):
        pltpu.sync_copy(rows_v.at[q % 2],
                        acc_sh.at[pl.ds(s * ZR + q * GEDGES, GEDGES)])
    pltpu.sync_copy(rows_v.at[0, pl.ds(0, ZR - 3 * GEDGES)],
                    acc_sh.at[pl.ds(s * ZR + 3 * GEDGES, ZR - 3 * GEDGES)])
    if with_deg:
        for q in range(3):
            pltpu.sync_copy(ones_v,
                            deg_sh.at[pl.ds(s * ZR + q * GEDGES, GEDGES)])
        pltpu.sync_copy(ones_v.at[pl.ds(0, ZR - 3 * GEDGES)],
                        deg_sh.at[pl.ds(s * ZR + 3 * GEDGES, ZR - 3 * GEDGES)])
        ov = jnp.ones((16,), jnp.float32)

        def orow(i, carry):
            ones_v[i] = ov
            return carry

        lax.fori_loop(0, GEDGES, orow, 0)

    pltpu.make_async_copy(edge_hbm.at[0, pl.ds(base, EPW)], src_v,
                          isem).wait()
    for g in range(NGRP):
        pltpu.make_async_copy(edge_hbm.at[1, pl.ds(base + g * GEDGES, GEDGES)],
                              dst_v.at[g, 0], gsem.at[0]).wait()
    plsc.subcore_barrier()

    def gather(r, g):
        pltpu.async_copy(feat_hbm.at[src_v.at[pl.ds(g * GEDGES, GEDGES)]],
                         rows_v.at[r], gsem.at[r])

    gather(0, 0)

    def pair(p, carry):
        for r in range(2):
            g = 2 * p + r
            pltpu.make_async_copy(
                feat_hbm.at[src_v.at[pl.ds(g * GEDGES, GEDGES)]],
                rows_v.at[r], gsem.at[r]).wait()

            @pl.when(g + 1 < NGRP)
            def _():
                gather(1 - r, g + 1)

            pltpu.sync_copy(rows_v.at[r], acc_sh.at[dst_v.at[g, 0]], add=True)
            if with_deg:
                pltpu.sync_copy(ones_v, deg_sh.at[dst_v.at[g, 0]], add=True)
        return carry

    lax.fori_loop(0, NGRP // 2, pair, 0)
    plsc.subcore_barrier()

    @pl.when(s < NS - 1)
    def _():
        pltpu.sync_copy(acc_sh.at[pl.ds(s * RPT, RPT)],
                        acc_out.at[c, pl.ds(s * RPT, RPT)])
        if with_deg:
            pltpu.sync_copy(deg_sh.at[pl.ds(s * RPT, RPT)],
                            deg_out.at[c, pl.ds(s * RPT, RPT)])

    @pl.when(s == NS - 1)
    def _():
        last = N - (NS - 1) * RPT
        pltpu.sync_copy(acc_sh.at[pl.ds((NS - 1) * RPT, last)],
                        acc_out.at[c, pl.ds((NS - 1) * RPT, last)])
        if with_deg:
            pltpu.sync_copy(deg_sh.at[pl.ds((NS - 1) * RPT, last)],
                            deg_out.at[c, pl.ds((NS - 1) * RPT, last)])


_sc_agg_deg = functools.partial(
    pl.kernel,
    out_type=(jax.ShapeDtypeStruct((NC, N, HID), jnp.float32),
              jax.ShapeDtypeStruct((NC, N, 16), jnp.float32)),
    mesh=_MESH,
    scratch_types=[
        pltpu.VMEM((EPW,), jnp.int32),
        pltpu.VMEM((NGRP, 1, GEDGES), jnp.int32),
        pltpu.VMEM((2, GEDGES, HID), jnp.float32),
        pltpu.VMEM((GEDGES, 16), jnp.float32),
        pltpu.VMEM_SHARED((ACC_N, HID), jnp.float32),
        pltpu.VMEM_SHARED((ACC_N, 16), jnp.float32),
        pltpu.SemaphoreType.DMA((2,)),
        pltpu.SemaphoreType.DMA,
    ],
    compiler_params=_SC_PARAMS,
)(functools.partial(_sc_body, True))


def _sc_body_nodeg(feat_hbm, edge_hbm, acc_out,
                   src_v, dst_v, rows_v, acc_sh, gsem, isem):
    _sc_body(False, feat_hbm, edge_hbm,
             acc_out, None, src_v, dst_v, rows_v, None,
             acc_sh, None, gsem, isem)


_sc_agg = functools.partial(
    pl.kernel,
    out_type=jax.ShapeDtypeStruct((NC, N, HID), jnp.float32),
    mesh=_MESH,
    scratch_types=[
        pltpu.VMEM((EPW,), jnp.int32),
        pltpu.VMEM((NGRP, 1, GEDGES), jnp.int32),
        pltpu.VMEM((2, GEDGES, HID), jnp.float32),
        pltpu.VMEM_SHARED((ACC_N, HID), jnp.float32),
        pltpu.SemaphoreType.DMA((2,)),
        pltpu.SemaphoreType.DMA,
    ],
    compiler_params=_SC_PARAMS,
)(_sc_body_nodeg)


RB = 2000
GRID = N // RB


def _mm_body(x_ref, w_ref, o_ref):
    o_ref[...] = jnp.dot(x_ref[...], w_ref[...],
                         preferred_element_type=jnp.float32)


def _tc_mm(x, w, kdim, ndim):
    return pl.pallas_call(
        _mm_body,
        grid=(GRID,),
        in_specs=[
            pl.BlockSpec((RB, kdim), lambda i: (i, 0)),
            pl.BlockSpec((kdim, ndim), lambda i: (0, 0)),
        ],
        out_specs=pl.BlockSpec((RB, ndim), lambda i: (i, 0)),
        out_shape=jax.ShapeDtypeStruct((N, ndim), jnp.float32),
    )(x, w)


def _combine(acc_ref, deg_ref):
    agg = acc_ref[0] + acc_ref[1]
    deg = deg_ref[0, :, 0:1] + deg_ref[1, :, 0:1]
    return agg / jnp.maximum(deg, 1.0)


def _layer_mid_body(s_ref, acc_ref, deg_ref, b_ref, wn_ref,
                    oh_ref, op_ref):
    h = jnp.maximum(s_ref[...] + _combine(acc_ref, deg_ref) + b_ref[...], 0.0)
    oh_ref[...] = h
    op_ref[...] = jnp.dot(h, wn_ref[...], preferred_element_type=jnp.float32)


def _tc_layer_mid(sprev, acc, deg, b, wn):
    return pl.pallas_call(
        _layer_mid_body,
        grid=(GRID,),
        in_specs=[
            pl.BlockSpec((RB, HID), lambda i: (i, 0)),
            pl.BlockSpec((NC, RB, HID), lambda i: (0, i, 0)),
            pl.BlockSpec((NC, RB, 16), lambda i: (0, i, 0)),
            pl.BlockSpec((1, HID), lambda i: (0, 0)),
            pl.BlockSpec((HID, HID), lambda i: (0, 0)),
        ],
        out_specs=[
            pl.BlockSpec((RB, HID), lambda i: (i, 0)),
            pl.BlockSpec((RB, HID), lambda i: (i, 0)),
        ],
        out_shape=[jax.ShapeDtypeStruct((N, HID), jnp.float32),
                   jax.ShapeDtypeStruct((N, HID), jnp.float32)],
    )(sprev, acc, deg, b, wn)


def _layer2_body(s_ref, acc_ref, deg_ref, b_ref, oh_ref):
    oh_ref[...] = jnp.maximum(
        s_ref[...] + _combine(acc_ref, deg_ref) + b_ref[...], 0.0)


def _tc_layer2(sprev, acc, deg, b):
    return pl.pallas_call(
        _layer2_body,
        grid=(GRID,),
        in_specs=[
            pl.BlockSpec((RB, HID), lambda i: (i, 0)),
            pl.BlockSpec((NC, RB, HID), lambda i: (0, i, 0)),
            pl.BlockSpec((NC, RB, 16), lambda i: (0, i, 0)),
            pl.BlockSpec((1, HID), lambda i: (0, 0)),
        ],
        out_specs=pl.BlockSpec((RB, HID), lambda i: (i, 0)),
        out_shape=jax.ShapeDtypeStruct((N, HID), jnp.float32),
    )(sprev, acc, deg, b)


def _layer3_body(s_ref, acc_ref, deg_ref, b_ref, wn_ref, o_ref):
    hn = _combine(acc_ref, deg_ref)
    o_ref[...] = (s_ref[...] + b_ref[...]
                  + jnp.dot(hn, wn_ref[...], preferred_element_type=jnp.float32))


def _tc_layer3(sprev, acc, deg, b, wn):
    return pl.pallas_call(
        _layer3_body,
        grid=(GRID,),
        in_specs=[
            pl.BlockSpec((RB, OUT), lambda i: (i, 0)),
            pl.BlockSpec((NC, RB, HID), lambda i: (0, i, 0)),
            pl.BlockSpec((NC, RB, 16), lambda i: (0, i, 0)),
            pl.BlockSpec((1, OUT), lambda i: (0, 0)),
            pl.BlockSpec((HID, OUT), lambda i: (0, 0)),
        ],
        out_specs=pl.BlockSpec((RB, OUT), lambda i: (i, 0)),
        out_shape=jax.ShapeDtypeStruct((N, OUT), jnp.float32),
    )(sprev, acc, deg, b, wn)


def kernel(x, edge_index, W_self1, W_neigh1, b1, W_self2, W_neigh2, b2,
           W_self3, W_neigh3, b3):
    edges = edge_index.astype(jnp.int32)

    p1 = _tc_mm(x, W_neigh1, IN, HID)
    acc1, deg = _sc_agg_deg(p1, edges)
    s1 = _tc_mm(x, W_self1, IN, HID)
    h1, p2 = _tc_layer_mid(s1, acc1, deg, b1.reshape(1, HID), W_neigh2)
    acc2 = _sc_agg(p2, edges)
    s2 = _tc_mm(h1, W_self2, HID, HID)
    h2 = _tc_layer2(s2, acc2, deg, b2.reshape(1, HID))
    acc3 = _sc_agg(h2, edges)
    s3 = _tc_mm(h2, W_self3, HID, OUT)
    return _tc_layer3(s3, acc3, deg, b3.reshape(1, OUT), W_neigh3)
